# baseline (speedup 1.0000x reference)
import functools

import jax
import jax.numpy as jnp
from jax import lax
from jax.experimental import pallas as pl
from jax.experimental.pallas import tpu as pltpu
from jax.experimental.pallas import tpu_sc as plsc

N = 10000
NP = 10240
E = 320000
H = 128
G = 64
NCORES = 2
NSUB = 16
NTILES = NCORES * NSUB
CH = 80
K = 128
EP = NTILES * CH * K
RPT = NP // NSUB
DUMMY = N
R = 1024

_mesh = plsc.VectorSubcoreMesh(core_axis_name="c", subcore_axis_name="s",
                               num_cores=NCORES, num_subcores=NSUB)


@functools.partial(
    pl.kernel,
    out_type=jax.ShapeDtypeStruct((NCORES, NP), jnp.float32),
    mesh=_mesh,
    scratch_types=[
        pltpu.VMEM_SHARED((NP,), jnp.float32),
        pltpu.VMEM((CH, K), jnp.int32),
        pltpu.VMEM((K,), jnp.float32),
        pltpu.VMEM((RPT,), jnp.float32),
    ],
)
def _deg_pass(dstp, dparts, dacc, didx, ones, zeros):
    c = lax.axis_index("c")
    s = lax.axis_index("s")
    tid = c * NSUB + s
    for k in range(K // 16):
        ones[pl.ds(k * 16, 16)] = jnp.ones((16,), jnp.float32)
    for k in range(RPT // 16):
        zeros[pl.ds(k * 16, 16)] = jnp.zeros((16,), jnp.float32)
    pltpu.sync_copy(zeros, dacc.at[pl.ds(s * RPT, RPT)])
    pltpu.sync_copy(dstp.at[tid], didx)
    plsc.subcore_barrier()

    def chunk(i, carry):
        pltpu.sync_copy(ones, dacc.at[didx.at[i]], add=True)
        return carry

    lax.fori_loop(0, CH, chunk, 0)
    plsc.subcore_barrier()
    pltpu.sync_copy(dacc.at[pl.ds(s * RPT, RPT)], dparts.at[c, pl.ds(s * RPT, RPT)])


CHF = 160
NCHUNK = NSUB * CHF
SLABF = 40
ACCR = NP // NSUB


@functools.partial(
    pl.kernel,
    out_type=jax.ShapeDtypeStruct((NP, H), jnp.float32),
    mesh=_mesh,
    scratch_types=[
        pltpu.VMEM_SHARED((NP, H), jnp.float32),
        pltpu.VMEM((SLABF, 2, K), jnp.int32),
        pltpu.VMEM((K, H), jnp.float32),
        pltpu.VMEM((K, H), jnp.float32),
        pltpu.SemaphoreType.DMA,
        pltpu.SemaphoreType.DMA,
    ],
)
def _edge_pass(y, idxp, parts, acc, idxb, rows0, rows1, sg, ss):
    c = lax.axis_index("c")
    s = lax.axis_index("s")

    @pl.when(c == 0)
    def _():
        pltpu.sync_copy(y.at[pl.ds(s * ACCR, ACCR)],
                        acc.at[pl.ds(s * ACCR, ACCR)])

    plsc.subcore_barrier()

    def run(base0, nch, slab):
        for h in range(nch // slab):
            base = base0 + h * slab
            pltpu.sync_copy(idxp.at[pl.ds(base, slab)],
                            idxb.at[pl.ds(0, slab)])
            pltpu.async_copy(y.at[idxb.at[0, 0]], rows0, sg)

            def pair(g, carry):
                i0 = 2 * g
                pltpu.make_async_copy(y.at[idxb.at[i0, 0]], rows0, sg).wait()

                @pl.when(g > 0)
                def _():
                    pltpu.make_async_copy(rows1, acc.at[idxb.at[0, 1]], ss).wait()

                pltpu.async_copy(y.at[idxb.at[i0 + 1, 0]], rows1, sg)
                pltpu.async_copy(rows0, acc.at[idxb.at[i0, 1]], ss, add=True)
                pltpu.make_async_copy(y.at[idxb.at[i0 + 1, 0]], rows1, sg).wait()
                pltpu.make_async_copy(rows0, acc.at[idxb.at[0, 1]], ss).wait()

                @pl.when(i0 + 2 < slab)
                def _():
                    pltpu.async_copy(y.at[idxb.at[i0 + 2, 0]], rows0, sg)

                pltpu.async_copy(rows1, acc.at[idxb.at[i0 + 1, 1]], ss, add=True)
                return carry

            lax.fori_loop(0, slab // 2, pair, 0)
            pltpu.make_async_copy(rows1, acc.at[idxb.at[0, 1]], ss).wait()

    @pl.when(c == 0)
    def _():
        run(s * CHF, CHF, SLABF)

    plsc.subcore_barrier()

    @pl.when(c == 0)
    def _():
        pltpu.sync_copy(acc.at[pl.ds(s * ACCR, ACCR)],
                        parts.at[pl.ds(s * ACCR, ACCR)])


def _prologue_body(x_ref, dp_ref, w_ref, y_ref, dinv_ref):
    deg = dp_ref[0] + dp_ref[1] + 1.0
    dinv = lax.rsqrt(deg)
    dinv_ref[...] = dinv
    y_ref[...] = dinv * jnp.dot(x_ref[...], w_ref[...],
                                preferred_element_type=jnp.float32)


_prologue = pl.pallas_call(
    _prologue_body,
    grid=(NP // R,),
    in_specs=[
        pl.BlockSpec((R, H), lambda i: (i, 0)),
        pl.BlockSpec((NCORES, R, 1), lambda i: (0, i, 0)),
        pl.BlockSpec((H, H), lambda i: (0, 0)),
    ],
    out_specs=[
        pl.BlockSpec((R, H), lambda i: (i, 0)),
        pl.BlockSpec((R, 1), lambda i: (i, 0)),
    ],
    out_shape=[
        jax.ShapeDtypeStruct((NP, H), jnp.float32),
        jax.ShapeDtypeStruct((NP, 1), jnp.float32),
    ],
)


def _combine_mm_body(p_ref, dinv_ref, b_ref, w_ref, yn_ref):
    dinv = dinv_ref[...]
    h = jnp.maximum(dinv * p_ref[...] + b_ref[...], 0.0)
    yn_ref[...] = dinv * jnp.dot(h, w_ref[...],
                                 preferred_element_type=jnp.float32)


_combine_mm = pl.pallas_call(
    _combine_mm_body,
    grid=(NP // R,),
    in_specs=[
        pl.BlockSpec((R, H), lambda i: (i, 0)),
        pl.BlockSpec((R, 1), lambda i: (i, 0)),
        pl.BlockSpec((1, H), lambda i: (0, 0)),
        pl.BlockSpec((H, H), lambda i: (0, 0)),
    ],
    out_specs=pl.BlockSpec((R, H), lambda i: (i, 0)),
    out_shape=jax.ShapeDtypeStruct((NP, H), jnp.float32),
)


def _combine_last_body(p_ref, dinv_ref, b_ref, h_ref):
    i = pl.program_id(0)
    row = lax.broadcasted_iota(jnp.int32, (R, 1), 0) + i * R
    h = jnp.maximum(dinv_ref[...] * p_ref[...] + b_ref[...], 0.0)
    h_ref[...] = jnp.where(row < N, h, 0.0)


_combine_last = pl.pallas_call(
    _combine_last_body,
    grid=(NP // R,),
    in_specs=[
        pl.BlockSpec((R, H), lambda i: (i, 0)),
        pl.BlockSpec((R, 1), lambda i: (i, 0)),
        pl.BlockSpec((1, H), lambda i: (0, 0)),
    ],
    out_specs=pl.BlockSpec((R, H), lambda i: (i, 0)),
    out_shape=jax.ShapeDtypeStruct((NP, H), jnp.float32),
)


def _head_body(h_ref, bt_ref, wf_ref, bf_ref, out_ref, mx_ref):
    h = h_ref[...]
    bt = bt_ref[...]
    gid = lax.broadcasted_iota(jnp.int32, (NP, G), 1)
    m = (bt == gid).astype(jnp.float32)
    sums = lax.dot_general(m, h, (((0,), (0,)), ((), ())),
                           preferred_element_type=jnp.float32)
    cnt = lax.dot_general(m, jnp.ones((NP, 1), jnp.float32),
                          (((0,), (0,)), ((), ())),
                          preferred_element_type=jnp.float32)
    mean = sums / jnp.maximum(cnt, 1.0)

    def mbody(g, carry):
        mg = jnp.max(jnp.where(bt == g, h, -jnp.inf), axis=0, keepdims=True)
        mx_ref[pl.ds(g, 1), :] = mg
        return carry

    lax.fori_loop(0, G, mbody, 0)
    pooled = jnp.concatenate([mean, mx_ref[...]], axis=1)
    logits = jnp.dot(pooled, wf_ref[...],
                     preferred_element_type=jnp.float32) + bf_ref[...]
    zmax = jnp.max(logits, axis=1, keepdims=True)
    zs = logits - zmax
    out_ref[...] = zs - jnp.log(jnp.sum(jnp.exp(zs), axis=1, keepdims=True))


_head = pl.pallas_call(
    _head_body,
    out_shape=jax.ShapeDtypeStruct((G, 6), jnp.float32),
    scratch_shapes=[pltpu.VMEM((G, H), jnp.float32)],
)


def kernel(x, edge_index, batch, W1, b1, W2, b2, W3, b3, W4, b4, Wf, bf):
    f32 = jnp.float32
    i32 = jnp.int32
    pad_e = EP - E
    src = jnp.concatenate(
        [edge_index[0], jnp.zeros((pad_e,), i32)]).reshape(NCHUNK, 1, K)
    dst = jnp.concatenate(
        [edge_index[1], jnp.full((pad_e,), DUMMY, i32)]).reshape(NCHUNK, 1, K)
    idxp = jnp.concatenate([src, dst], axis=1)
    dstt = dst.reshape(NTILES, CH, K)
    x_pad = jnp.concatenate([x, jnp.zeros((NP - N, H), f32)], axis=0)
    batch2d = jnp.concatenate(
        [batch, jnp.full((NP - N,), G, i32)]).reshape(NP, 1)

    dparts = _deg_pass(dstt).reshape(NCORES, NP, 1)
    y, dinv = _prologue(x_pad, dparts, W1)
    for b_l, W_next in ((b1, W2), (b2, W3), (b3, W4)):
        p = _edge_pass(y, idxp)
        y = _combine_mm(p, dinv, b_l.reshape(1, H), W_next)
    p = _edge_pass(y, idxp)
    h4 = _combine_last(p, dinv, b4.reshape(1, H))
    return _head(h4, batch2d, Wf, bf.reshape(1, 6))


# speedup vs baseline: 2.1055x; 2.1055x over previous
import functools

import jax
import jax.numpy as jnp
from jax import lax
from jax.experimental import pallas as pl
from jax.experimental.pallas import tpu as pltpu
from jax.experimental.pallas import tpu_sc as plsc

N = 10000
NP = 10240
E = 320000
H = 128
G = 64
NCORES = 2
NSUB = 16
NTILES = NCORES * NSUB
CH = 80
K = 128
EP = NTILES * CH * K
RPT = NP // NSUB
DUMMY = N
R = 1024

_mesh = plsc.VectorSubcoreMesh(core_axis_name="c", subcore_axis_name="s",
                               num_cores=NCORES, num_subcores=NSUB)


@functools.partial(
    pl.kernel,
    out_type=jax.ShapeDtypeStruct((NCORES, NP), jnp.float32),
    mesh=_mesh,
    scratch_types=[
        pltpu.VMEM_SHARED((NP,), jnp.float32),
        pltpu.VMEM((CH, K), jnp.int32),
        pltpu.VMEM((K,), jnp.float32),
        pltpu.VMEM((RPT,), jnp.float32),
    ],
)
def _deg_pass(dstp, dparts, dacc, didx, ones, zeros):
    c = lax.axis_index("c")
    s = lax.axis_index("s")
    tid = c * NSUB + s
    for k in range(K // 16):
        ones[pl.ds(k * 16, 16)] = jnp.ones((16,), jnp.float32)
    for k in range(RPT // 16):
        zeros[pl.ds(k * 16, 16)] = jnp.zeros((16,), jnp.float32)
    pltpu.sync_copy(zeros, dacc.at[pl.ds(s * RPT, RPT)])
    pltpu.sync_copy(dstp.at[tid], didx)
    plsc.subcore_barrier()

    def chunk(i, carry):
        pltpu.sync_copy(ones, dacc.at[didx.at[i]], add=True)
        return carry

    lax.fori_loop(0, CH, chunk, 0)
    plsc.subcore_barrier()
    pltpu.sync_copy(dacc.at[pl.ds(s * RPT, RPT)], dparts.at[c, pl.ds(s * RPT, RPT)])


CHF = 160
NCHUNK = NSUB * CHF
SLABF = 40
ACCR = NP // NSUB


@functools.partial(
    pl.kernel,
    out_type=jax.ShapeDtypeStruct((NP, H), jnp.float32),
    mesh=_mesh,
    scratch_types=[
        pltpu.VMEM_SHARED((NP, H), jnp.float32),
        pltpu.VMEM((SLABF, 2, K), jnp.int32),
        pltpu.VMEM((K, H), jnp.float32),
        pltpu.VMEM((K, H), jnp.float32),
        pltpu.SemaphoreType.DMA,
        pltpu.SemaphoreType.DMA,
    ],
)
def _edge_pass(y, idxp, parts, acc, idxb, rows0, rows1, sg, ss):
    c = lax.axis_index("c")
    s = lax.axis_index("s")

    @pl.when(c == 0)
    def _():
        pltpu.sync_copy(y.at[pl.ds(s * ACCR, ACCR)],
                        acc.at[pl.ds(s * ACCR, ACCR)])

    plsc.subcore_barrier()

    def run(base0, nch, slab):
        for h in range(nch // slab):
            base = base0 + h * slab
            pltpu.sync_copy(idxp.at[pl.ds(base, slab)],
                            idxb.at[pl.ds(0, slab)])
            pltpu.async_copy(y.at[idxb.at[0, 0]], rows0, sg)

            def pair(g, carry):
                i0 = 2 * g
                pltpu.make_async_copy(y.at[idxb.at[i0, 0]], rows0, sg).wait()

                @pl.when(g > 0)
                def _():
                    pltpu.make_async_copy(rows1, acc.at[idxb.at[0, 1]], ss).wait()

                pltpu.async_copy(y.at[idxb.at[i0 + 1, 0]], rows1, sg)
                pltpu.async_copy(rows0, acc.at[idxb.at[i0, 1]], ss, add=True)
                pltpu.make_async_copy(y.at[idxb.at[i0 + 1, 0]], rows1, sg).wait()
                pltpu.make_async_copy(rows0, acc.at[idxb.at[0, 1]], ss).wait()

                @pl.when(i0 + 2 < slab)
                def _():
                    pltpu.async_copy(y.at[idxb.at[i0 + 2, 0]], rows0, sg)

                pltpu.async_copy(rows1, acc.at[idxb.at[i0 + 1, 1]], ss, add=True)
                return carry

            lax.fori_loop(0, slab // 2, pair, 0)
            pltpu.make_async_copy(rows1, acc.at[idxb.at[0, 1]], ss).wait()

    @pl.when(c == 0)
    def _():
        run(s * CHF, CHF, SLABF)

    plsc.subcore_barrier()

    @pl.when(c == 0)
    def _():
        pltpu.sync_copy(acc.at[pl.ds(s * ACCR, ACCR)],
                        parts.at[pl.ds(s * ACCR, ACCR)])


def _prologue_body(x_ref, dp_ref, w_ref, y_ref, dinv_ref):
    deg = dp_ref[0] + dp_ref[1] + 1.0
    dinv = lax.rsqrt(deg)
    dinv_ref[...] = dinv
    y_ref[...] = dinv * jnp.dot(x_ref[...], w_ref[...],
                                preferred_element_type=jnp.float32)


_prologue = pl.pallas_call(
    _prologue_body,
    grid=(NP // R,),
    in_specs=[
        pl.BlockSpec((R, H), lambda i: (i, 0)),
        pl.BlockSpec((NCORES, R, 1), lambda i: (0, i, 0)),
        pl.BlockSpec((H, H), lambda i: (0, 0)),
    ],
    out_specs=[
        pl.BlockSpec((R, H), lambda i: (i, 0)),
        pl.BlockSpec((R, 1), lambda i: (i, 0)),
    ],
    out_shape=[
        jax.ShapeDtypeStruct((NP, H), jnp.float32),
        jax.ShapeDtypeStruct((NP, 1), jnp.float32),
    ],
)


def _combine_mm_body(p_ref, dinv_ref, b_ref, w_ref, yn_ref):
    dinv = dinv_ref[...]
    h = jnp.maximum(dinv * p_ref[...] + b_ref[...], 0.0)
    yn_ref[...] = dinv * jnp.dot(h, w_ref[...],
                                 preferred_element_type=jnp.float32)


_combine_mm = pl.pallas_call(
    _combine_mm_body,
    grid=(NP // R,),
    in_specs=[
        pl.BlockSpec((R, H), lambda i: (i, 0)),
        pl.BlockSpec((R, 1), lambda i: (i, 0)),
        pl.BlockSpec((1, H), lambda i: (0, 0)),
        pl.BlockSpec((H, H), lambda i: (0, 0)),
    ],
    out_specs=pl.BlockSpec((R, H), lambda i: (i, 0)),
    out_shape=jax.ShapeDtypeStruct((NP, H), jnp.float32),
)


def _combine_last_body(p_ref, dinv_ref, b_ref, h_ref):
    i = pl.program_id(0)
    row = lax.broadcasted_iota(jnp.int32, (R, 1), 0) + i * R
    h = jnp.maximum(dinv_ref[...] * p_ref[...] + b_ref[...], 0.0)
    h_ref[...] = jnp.where(row < N, h, 0.0)


_combine_last = pl.pallas_call(
    _combine_last_body,
    grid=(NP // R,),
    in_specs=[
        pl.BlockSpec((R, H), lambda i: (i, 0)),
        pl.BlockSpec((R, 1), lambda i: (i, 0)),
        pl.BlockSpec((1, H), lambda i: (0, 0)),
    ],
    out_specs=pl.BlockSpec((R, H), lambda i: (i, 0)),
    out_shape=jax.ShapeDtypeStruct((NP, H), jnp.float32),
)


def _head_body(h_ref, bt_ref, wf_ref, bf_ref, out_ref, mx_ref):
    h = h_ref[...]
    bt = bt_ref[...]
    gid = lax.broadcasted_iota(jnp.int32, (NP, G), 1)
    m = (bt == gid).astype(jnp.float32)
    sums = lax.dot_general(m, h, (((0,), (0,)), ((), ())),
                           preferred_element_type=jnp.float32)
    cnt = lax.dot_general(m, jnp.ones((NP, 1), jnp.float32),
                          (((0,), (0,)), ((), ())),
                          preferred_element_type=jnp.float32)
    mean = sums / jnp.maximum(cnt, 1.0)

    def mbody(g, carry):
        mg = jnp.max(jnp.where(bt == g, h, -jnp.inf), axis=0, keepdims=True)
        mx_ref[pl.ds(g, 1), :] = mg
        return carry

    lax.fori_loop(0, G, mbody, 0)
    pooled = jnp.concatenate([mean, mx_ref[...]], axis=1)
    logits = jnp.dot(pooled, wf_ref[...],
                     preferred_element_type=jnp.float32) + bf_ref[...]
    zmax = jnp.max(logits, axis=1, keepdims=True)
    zs = logits - zmax
    out_ref[...] = zs - jnp.log(jnp.sum(jnp.exp(zs), axis=1, keepdims=True))


_head = pl.pallas_call(
    _head_body,
    out_shape=jax.ShapeDtypeStruct((G, 6), jnp.float32),
    scratch_shapes=[pltpu.VMEM((G, H), jnp.float32)],
)


def kernel(x, edge_index, batch, W1, b1, W2, b2, W3, b3, W4, b4, Wf, bf):
    f32 = jnp.float32
    i32 = jnp.int32
    pad_e = EP - E
    pad_i = jnp.arange(pad_e, dtype=i32)
    src = jnp.concatenate(
        [edge_index[0], pad_i % N]).reshape(NCHUNK, 1, K)
    dst = jnp.concatenate(
        [edge_index[1], DUMMY + pad_i % (NP - N)]).reshape(NCHUNK, 1, K)
    idxp = jnp.concatenate([src, dst], axis=1)
    dstt = dst.reshape(NTILES, CH, K)
    x_pad = jnp.concatenate([x, jnp.zeros((NP - N, H), f32)], axis=0)
    batch2d = jnp.concatenate(
        [batch, jnp.full((NP - N,), G, i32)]).reshape(NP, 1)

    dparts = _deg_pass(dstt).reshape(NCORES, NP, 1)
    y, dinv = _prologue(x_pad, dparts, W1)
    for b_l, W_next in ((b1, W2), (b2, W3), (b3, W4)):
        p = _edge_pass(y, idxp)
        y = _combine_mm(p, dinv, b_l.reshape(1, H), W_next)
    p = _edge_pass(y, idxp)
    h4 = _combine_last(p, dinv, b4.reshape(1, H))
    return _head(h4, batch2d, Wf, bf.reshape(1, 6))


# speedup vs baseline: 2.2021x; 1.0459x over previous
import functools

import jax
import jax.numpy as jnp
from jax import lax
from jax.experimental import pallas as pl
from jax.experimental.pallas import tpu as pltpu
from jax.experimental.pallas import tpu_sc as plsc

N = 10000
NP = 10240
E = 320000
H = 128
G = 64
NCORES = 2
NSUB = 16
NTILES = NCORES * NSUB
CH = 80
K = 128
EP = NTILES * CH * K
RPT = NP // NSUB
DUMMY = N
R = 1024

_mesh = plsc.VectorSubcoreMesh(core_axis_name="c", subcore_axis_name="s",
                               num_cores=NCORES, num_subcores=NSUB)


@functools.partial(
    pl.kernel,
    out_type=jax.ShapeDtypeStruct((NCORES, NP), jnp.float32),
    mesh=_mesh,
    scratch_types=[
        pltpu.VMEM_SHARED((NP,), jnp.float32),
        pltpu.VMEM((CH, K), jnp.int32),
        pltpu.VMEM((K,), jnp.float32),
        pltpu.VMEM((RPT,), jnp.float32),
    ],
)
def _deg_pass(dstp, dparts, dacc, didx, ones, zeros):
    c = lax.axis_index("c")
    s = lax.axis_index("s")
    tid = c * NSUB + s
    for k in range(K // 16):
        ones[pl.ds(k * 16, 16)] = jnp.ones((16,), jnp.float32)
    for k in range(RPT // 16):
        zeros[pl.ds(k * 16, 16)] = jnp.zeros((16,), jnp.float32)
    pltpu.sync_copy(zeros, dacc.at[pl.ds(s * RPT, RPT)])
    pltpu.sync_copy(dstp.at[tid], didx)
    plsc.subcore_barrier()

    def chunk(i, carry):
        pltpu.sync_copy(ones, dacc.at[didx.at[i]], add=True)
        return carry

    lax.fori_loop(0, CH, chunk, 0)
    plsc.subcore_barrier()
    pltpu.sync_copy(dacc.at[pl.ds(s * RPT, RPT)], dparts.at[c, pl.ds(s * RPT, RPT)])


CHF = 160
NCHUNK = NSUB * CHF
SLABF = 40
ACCR = NP // NSUB


@functools.partial(
    pl.kernel,
    out_type=jax.ShapeDtypeStruct((NP, H), jnp.float32),
    mesh=_mesh,
    scratch_types=[
        pltpu.VMEM_SHARED((NP, H), jnp.float32),
        pltpu.VMEM((SLABF, 2, K), jnp.int32),
        pltpu.VMEM((K, H), jnp.float32),
        pltpu.VMEM((K, H), jnp.float32),
        pltpu.SemaphoreType.DMA,
        pltpu.SemaphoreType.DMA,
    ],
)
def _edge_pass(y, idxp, parts, acc, idxb, rows0, rows1, sg, ss):
    c = lax.axis_index("c")
    s = lax.axis_index("s")

    @pl.when(c == 0)
    def _():
        pltpu.sync_copy(y.at[pl.ds(s * ACCR, ACCR)],
                        acc.at[pl.ds(s * ACCR, ACCR)])

    plsc.subcore_barrier()

    def run(base0, nch, slab):
        for h in range(nch // slab):
            base = base0 + h * slab
            pltpu.sync_copy(idxp.at[pl.ds(base, slab)],
                            idxb.at[pl.ds(0, slab)])
            pltpu.async_copy(y.at[idxb.at[0, 0]], rows0, sg)

            def pair(g, carry):
                i0 = 2 * g
                pltpu.make_async_copy(y.at[idxb.at[i0, 0]], rows0, sg).wait()

                @pl.when(g > 0)
                def _():
                    pltpu.make_async_copy(rows1, acc.at[idxb.at[0, 1]], ss).wait()

                pltpu.async_copy(y.at[idxb.at[i0 + 1, 0]], rows1, sg)
                pltpu.async_copy(rows0, acc.at[idxb.at[i0, 1]], ss, add=True)
                pltpu.make_async_copy(y.at[idxb.at[i0 + 1, 0]], rows1, sg).wait()
                pltpu.make_async_copy(rows0, acc.at[idxb.at[0, 1]], ss).wait()

                @pl.when(i0 + 2 < slab)
                def _():
                    pltpu.async_copy(y.at[idxb.at[i0 + 2, 0]], rows0, sg)

                pltpu.async_copy(rows1, acc.at[idxb.at[i0 + 1, 1]], ss, add=True)
                return carry

            lax.fori_loop(0, slab // 2, pair, 0)
            pltpu.make_async_copy(rows1, acc.at[idxb.at[0, 1]], ss).wait()

    @pl.when(c == 0)
    def _():
        run(s * CHF, CHF, SLABF)

    plsc.subcore_barrier()

    @pl.when(c == 0)
    def _():
        pltpu.sync_copy(acc.at[pl.ds(s * ACCR, ACCR)],
                        parts.at[pl.ds(s * ACCR, ACCR)])


def _prologue_body(x_ref, dp_ref, w_ref, y_ref, dinv_ref):
    deg = dp_ref[0] + dp_ref[1] + 1.0
    dinv = lax.rsqrt(deg)
    dinv_ref[...] = dinv
    y_ref[...] = dinv * jnp.dot(x_ref[...], w_ref[...],
                                preferred_element_type=jnp.float32)


_prologue = pl.pallas_call(
    _prologue_body,
    grid=(NP // R,),
    in_specs=[
        pl.BlockSpec((R, H), lambda i: (i, 0)),
        pl.BlockSpec((NCORES, R, 1), lambda i: (0, i, 0)),
        pl.BlockSpec((H, H), lambda i: (0, 0)),
    ],
    out_specs=[
        pl.BlockSpec((R, H), lambda i: (i, 0)),
        pl.BlockSpec((R, 1), lambda i: (i, 0)),
    ],
    out_shape=[
        jax.ShapeDtypeStruct((NP, H), jnp.float32),
        jax.ShapeDtypeStruct((NP, 1), jnp.float32),
    ],
)


def _combine_mm_body(p_ref, dinv_ref, b_ref, w_ref, yn_ref):
    dinv = dinv_ref[...]
    h = jnp.maximum(dinv * p_ref[...] + b_ref[...], 0.0)
    yn_ref[...] = dinv * jnp.dot(h, w_ref[...],
                                 preferred_element_type=jnp.float32)


_combine_mm = pl.pallas_call(
    _combine_mm_body,
    grid=(NP // R,),
    in_specs=[
        pl.BlockSpec((R, H), lambda i: (i, 0)),
        pl.BlockSpec((R, 1), lambda i: (i, 0)),
        pl.BlockSpec((1, H), lambda i: (0, 0)),
        pl.BlockSpec((H, H), lambda i: (0, 0)),
    ],
    out_specs=pl.BlockSpec((R, H), lambda i: (i, 0)),
    out_shape=jax.ShapeDtypeStruct((NP, H), jnp.float32),
)


def _combine_last_body(p_ref, dinv_ref, b_ref, h_ref):
    i = pl.program_id(0)
    row = lax.broadcasted_iota(jnp.int32, (R, 1), 0) + i * R
    h = jnp.maximum(dinv_ref[...] * p_ref[...] + b_ref[...], 0.0)
    h_ref[...] = jnp.where(row < N, h, 0.0)


_combine_last = pl.pallas_call(
    _combine_last_body,
    grid=(NP // R,),
    in_specs=[
        pl.BlockSpec((R, H), lambda i: (i, 0)),
        pl.BlockSpec((R, 1), lambda i: (i, 0)),
        pl.BlockSpec((1, H), lambda i: (0, 0)),
    ],
    out_specs=pl.BlockSpec((R, H), lambda i: (i, 0)),
    out_shape=jax.ShapeDtypeStruct((NP, H), jnp.float32),
)


GB = 128
NBLK = NP // GB


def _head_body(h_ref, bt_ref, wf_ref, bf_ref, out_ref, mx_ref, sm_ref, ct_ref):
    i = pl.program_id(0)

    @pl.when(i == 0)
    def _():
        mx_ref[...] = jnp.full((G, H), -jnp.inf, jnp.float32)
        sm_ref[...] = jnp.zeros((G, H), jnp.float32)
        ct_ref[...] = jnp.zeros((G, 1), jnp.float32)

    hblk = h_ref[...]
    bt = bt_ref[...]
    g0 = bt[0, 0]
    g1 = jnp.minimum(bt[GB - 1, 0], G - 1)

    def gbody(g, carry):
        mask = bt == g
        mg = jnp.max(jnp.where(mask, hblk, -jnp.inf), axis=0, keepdims=True)
        sg = jnp.sum(jnp.where(mask, hblk, 0.0), axis=0, keepdims=True)
        cg = jnp.sum(mask.astype(jnp.float32), axis=0, keepdims=True)
        mx_ref[pl.ds(g, 1), :] = jnp.maximum(mx_ref[pl.ds(g, 1), :], mg)
        sm_ref[pl.ds(g, 1), :] = sm_ref[pl.ds(g, 1), :] + sg
        ct_ref[pl.ds(g, 1), :] = ct_ref[pl.ds(g, 1), :] + cg
        return carry

    lax.fori_loop(g0, g1 + 1, gbody, 0)

    @pl.when(i == NBLK - 1)
    def _():
        mean = sm_ref[...] / jnp.maximum(ct_ref[...], 1.0)
        pooled = jnp.concatenate([mean, mx_ref[...]], axis=1)
        logits = jnp.dot(pooled, wf_ref[...],
                         preferred_element_type=jnp.float32) + bf_ref[...]
        zmax = jnp.max(logits, axis=1, keepdims=True)
        zs = logits - zmax
        out_ref[...] = zs - jnp.log(jnp.sum(jnp.exp(zs), axis=1, keepdims=True))


_head = pl.pallas_call(
    _head_body,
    grid=(NBLK,),
    in_specs=[
        pl.BlockSpec((GB, H), lambda i: (i, 0)),
        pl.BlockSpec((GB, 1), lambda i: (i, 0)),
        pl.BlockSpec((2 * H, 6), lambda i: (0, 0)),
        pl.BlockSpec((1, 6), lambda i: (0, 0)),
    ],
    out_specs=pl.BlockSpec((G, 6), lambda i: (0, 0)),
    out_shape=jax.ShapeDtypeStruct((G, 6), jnp.float32),
    scratch_shapes=[
        pltpu.VMEM((G, H), jnp.float32),
        pltpu.VMEM((G, H), jnp.float32),
        pltpu.VMEM((G, 1), jnp.float32),
    ],
)


def kernel(x, edge_index, batch, W1, b1, W2, b2, W3, b3, W4, b4, Wf, bf):
    f32 = jnp.float32
    i32 = jnp.int32
    pad_e = EP - E
    pad_i = jnp.arange(pad_e, dtype=i32)
    src = jnp.concatenate(
        [edge_index[0], pad_i % N]).reshape(NCHUNK, 1, K)
    dst = jnp.concatenate(
        [edge_index[1], DUMMY + pad_i % (NP - N)]).reshape(NCHUNK, 1, K)
    idxp = jnp.concatenate([src, dst], axis=1)
    dstt = dst.reshape(NTILES, CH, K)
    x_pad = jnp.concatenate([x, jnp.zeros((NP - N, H), f32)], axis=0)
    batch2d = jnp.concatenate(
        [batch, jnp.full((NP - N,), G, i32)]).reshape(NP, 1)

    dparts = _deg_pass(dstt).reshape(NCORES, NP, 1)
    y, dinv = _prologue(x_pad, dparts, W1)
    for b_l, W_next in ((b1, W2), (b2, W3), (b3, W4)):
        p = _edge_pass(y, idxp)
        y = _combine_mm(p, dinv, b_l.reshape(1, H), W_next)
    p = _edge_pass(y, idxp)
    h4 = _combine_last(p, dinv, b4.reshape(1, H))
    return _head(h4, batch2d, Wf, bf.reshape(1, 6))
